# Initial kernel scaffold; baseline (speedup 1.0000x reference)
import jax
import jax.numpy as jnp
from jax.experimental import pallas as pl


def kernel(x, edge_index, W1, b1, Wm, bm, W2, b2):
    raise NotImplementedError("write your pallas kernel here")


# speedup vs baseline: 11.0246x; 11.0246x over previous
import functools

import jax
import jax.numpy as jnp
from jax import lax
from jax.experimental import pallas as pl
from jax.experimental.pallas import tpu as pltpu
from jax.experimental.pallas import tpu_sc as plsc

N = 10000
E = 320000
D = 128

NC = 2
NS = 16
NW = NC * NS
EPW = E // NW
CH = 80
NCH = EPW // CH
NP = 10240
RPS = NP // NS

_MESH = plsc.VectorSubcoreMesh(core_axis_name="c", subcore_axis_name="s")


def _sc_degree(dst, zeros, ones):
    @functools.partial(
        pl.kernel,
        mesh=_MESH,
        out_type=jax.ShapeDtypeStruct((NC, NP, 128), jnp.float32),
        scratch_types=[
            pltpu.VMEM((CH,), jnp.int32),
            pltpu.VMEM((CH, 128), jnp.float32),
            pltpu.VMEM_SHARED((NP, 128), jnp.float32),
        ],
    )
    def deg_kernel(dst_hbm, zeros_hbm, ones_hbm, out_hbm, didx, ones_v, acc):
        c = lax.axis_index("c")
        s = lax.axis_index("s")
        pltpu.sync_copy(ones_hbm, ones_v)
        pltpu.sync_copy(zeros_hbm, acc.at[pl.ds(s * RPS, RPS)])
        plsc.subcore_barrier()
        base = (c * NS + s) * EPW

        def chunk(k, carry):
            pltpu.sync_copy(dst_hbm.at[pl.ds(base + k * CH, CH)], didx)
            pltpu.sync_copy(ones_v, acc.at[didx], add=True)
            return carry

        lax.fori_loop(0, NCH, chunk, 0)
        plsc.subcore_barrier()
        pltpu.sync_copy(acc.at[pl.ds(s * RPS, RPS)],
                        out_hbm.at[c, pl.ds(s * RPS, RPS)])

    return deg_kernel(dst, zeros, ones)


def _sc_aggregate(src, dst, hhat, zeros, feat):
    @functools.partial(
        pl.kernel,
        mesh=_MESH,
        out_type=jax.ShapeDtypeStruct((NC, NP, feat), jnp.float32),
        scratch_types=[
            pltpu.VMEM((CH,), jnp.int32),
            pltpu.VMEM((CH,), jnp.int32),
            pltpu.VMEM((CH, feat), jnp.float32),
            pltpu.VMEM_SHARED((NP, feat), jnp.float32),
            pltpu.SemaphoreType.DMA,
        ],
    )
    def agg_kernel(src_hbm, dst_hbm, h_hbm, zeros_hbm, out_hbm,
                   sidx, didx, rows, acc, sem):
        c = lax.axis_index("c")
        s = lax.axis_index("s")
        pltpu.sync_copy(zeros_hbm, acc.at[pl.ds(s * RPS, RPS)])
        plsc.subcore_barrier()
        base = (c * NS + s) * EPW

        def chunk(k, carry):
            off = base + k * CH
            pltpu.sync_copy(src_hbm.at[pl.ds(off, CH)], sidx)
            pltpu.async_copy(h_hbm.at[sidx], rows, sem).wait()
            pltpu.sync_copy(dst_hbm.at[pl.ds(off, CH)], didx)
            pltpu.sync_copy(rows, acc.at[didx], add=True)
            return carry

        lax.fori_loop(0, NCH, chunk, 0)
        plsc.subcore_barrier()
        pltpu.sync_copy(acc.at[pl.ds(s * RPS, RPS)],
                        out_hbm.at[c, pl.ds(s * RPS, RPS)])

    return agg_kernel(src, dst, hhat, zeros)


BN = 2000


def _tc_first(degp, x, w):
    def body(degp_ref, x_ref, w_ref, dis_ref, h_ref):
        deg = degp_ref[0, :, 0:1] + degp_ref[1, :, 0:1]
        dis = lax.rsqrt(deg + 1.0)
        dis_ref[...] = dis
        h = jnp.dot(x_ref[...], w_ref[...], preferred_element_type=jnp.float32)
        h_ref[...] = h * dis

    return pl.pallas_call(
        body,
        grid=(N // BN,),
        in_specs=[
            pl.BlockSpec((NC, BN, 128), lambda i: (0, i, 0)),
            pl.BlockSpec((BN, D), lambda i: (i, 0)),
            pl.BlockSpec(w.shape, lambda i: (0, 0)),
        ],
        out_specs=[
            pl.BlockSpec((BN, 1), lambda i: (i, 0)),
            pl.BlockSpec((BN, w.shape[1]), lambda i: (i, 0)),
        ],
        out_shape=[
            jax.ShapeDtypeStruct((N, 1), jnp.float32),
            jax.ShapeDtypeStruct((N, w.shape[1]), jnp.float32),
        ],
    )(degp, x, w)


def _tc_mid(aggp, hhat, dis, b, w):
    f_in = hhat.shape[1]
    f_out = w.shape[1]

    def body(aggp_ref, h_ref, dis_ref, b_ref, w_ref, o_ref):
        t = (aggp_ref[0] + aggp_ref[1] + h_ref[...]) * dis_ref[...] + b_ref[...]
        t = jnp.maximum(t, 0.0)
        o_ref[...] = jnp.dot(t, w_ref[...],
                             preferred_element_type=jnp.float32) * dis_ref[...]

    return pl.pallas_call(
        body,
        grid=(N // BN,),
        in_specs=[
            pl.BlockSpec((NC, BN, f_in), lambda i: (0, i, 0)),
            pl.BlockSpec((BN, f_in), lambda i: (i, 0)),
            pl.BlockSpec((BN, 1), lambda i: (i, 0)),
            pl.BlockSpec((1, f_in), lambda i: (0, 0)),
            pl.BlockSpec((f_in, f_out), lambda i: (0, 0)),
        ],
        out_specs=pl.BlockSpec((BN, f_out), lambda i: (i, 0)),
        out_shape=jax.ShapeDtypeStruct((N, f_out), jnp.float32),
    )(aggp, hhat, dis, b, w)


def _tc_final(aggp, hhat, dis, b):
    f = hhat.shape[1]

    def body(aggp_ref, h_ref, dis_ref, b_ref, o_ref):
        o_ref[...] = ((aggp_ref[0] + aggp_ref[1] + h_ref[...])
                      * dis_ref[...] + b_ref[...])

    return pl.pallas_call(
        body,
        grid=(N // BN,),
        in_specs=[
            pl.BlockSpec((NC, BN, f), lambda i: (0, i, 0)),
            pl.BlockSpec((BN, f), lambda i: (i, 0)),
            pl.BlockSpec((BN, 1), lambda i: (i, 0)),
            pl.BlockSpec((1, f), lambda i: (0, 0)),
        ],
        out_specs=pl.BlockSpec((BN, f), lambda i: (i, 0)),
        out_shape=jax.ShapeDtypeStruct((N, f), jnp.float32),
    )(aggp, hhat, dis, b)


def kernel(x, edge_index, W1, b1, Wm, bm, W2, b2):
    src = edge_index[0]
    dst = edge_index[1]
    H = W1.shape[1]
    C = W2.shape[1]
    CP = 128

    zerosH = jnp.zeros((RPS, H), jnp.float32)
    ones128 = jnp.ones((CH, 128), jnp.float32)
    w2p = jnp.zeros((H, CP), jnp.float32).at[:, :C].set(W2)
    b2p = jnp.zeros((CP,), jnp.float32).at[:C].set(b2)

    degp = _sc_degree(dst, zerosH, ones128)
    dis, h1 = _tc_first(degp, x, W1)

    a1 = _sc_aggregate(src, dst, h1, zerosH, H)
    h2 = _tc_mid(a1, h1, dis, b1.reshape(1, H), Wm)

    a2 = _sc_aggregate(src, dst, h2, zerosH, H)
    h3 = _tc_mid(a2, h2, dis, bm.reshape(1, H), w2p)

    a3 = _sc_aggregate(src, dst, h3, zerosH, CP)
    outp = _tc_final(a3, h3, dis, b2p.reshape(1, CP))
    return outp[:, :C]


# speedup vs baseline: 19.0112x; 1.7244x over previous
import functools

import jax
import jax.numpy as jnp
from jax import lax
from jax.experimental import pallas as pl
from jax.experimental.pallas import tpu as pltpu
from jax.experimental.pallas import tpu_sc as plsc

N = 10000
E = 320000
D = 128

NC = 2
NS = 16
NW = NC * NS
EPW = E // NW
CH = 40
NCH = EPW // CH
NP = 10240
RPS = NP // NS
NB = 5
NR = NCH // NB

_MESH = plsc.VectorSubcoreMesh(core_axis_name="c", subcore_axis_name="s")


def _sc_degree(dst, zeros, ones):
    @functools.partial(
        pl.kernel,
        mesh=_MESH,
        out_type=jax.ShapeDtypeStruct((NC, NP, 128), jnp.float32),
        scratch_types=(
            [pltpu.VMEM((CH,), jnp.int32)] * NB
            + [pltpu.VMEM((CH, 128), jnp.float32),
               pltpu.VMEM_SHARED((NP, 128), jnp.float32)]
            + [pltpu.SemaphoreType.DMA] * (2 * NB)
        ),
    )
    def deg_kernel(dst_hbm, zeros_hbm, ones_hbm, out_hbm, *sc):
        didx = sc[0:NB]
        ones_v = sc[NB]
        acc = sc[NB + 1]
        dsem = sc[NB + 2:2 * NB + 2]
        ssem = sc[2 * NB + 2:3 * NB + 2]
        c = lax.axis_index("c")
        s = lax.axis_index("s")
        pltpu.sync_copy(ones_hbm, ones_v)
        pltpu.sync_copy(zeros_hbm, acc.at[pl.ds(s * RPS, RPS)])
        plsc.subcore_barrier()
        base = (c * NS + s) * EPW

        for b in range(NB):
            pltpu.async_copy(dst_hbm.at[pl.ds(base + b * CH, CH)],
                             didx[b], dsem[b])

        def round_(j, carry):
            for b in range(NB):
                pltpu.make_async_copy(dst_hbm.at[pl.ds(base, CH)],
                                      didx[b], dsem[b]).wait()
                pltpu.async_copy(ones_v, acc.at[didx[b]], ssem[b], add=True)

            @pl.when(j < NR - 1)
            def _():
                for b in range(NB):
                    offn = base + ((j + 1) * NB + b) * CH
                    pltpu.make_async_copy(ones_v, acc.at[didx[b]],
                                          ssem[b]).wait()
                    pltpu.async_copy(dst_hbm.at[pl.ds(offn, CH)],
                                     didx[b], dsem[b])
            return carry

        lax.fori_loop(0, NR, round_, 0)
        for b in range(NB):
            pltpu.make_async_copy(ones_v, acc.at[didx[b]], ssem[b]).wait()
        plsc.subcore_barrier()
        pltpu.sync_copy(acc.at[pl.ds(s * RPS, RPS)],
                        out_hbm.at[c, pl.ds(s * RPS, RPS)])

    return deg_kernel(dst, zeros, ones)


def _sc_aggregate(src, dst, hhat, zeros, feat):
    @functools.partial(
        pl.kernel,
        mesh=_MESH,
        out_type=jax.ShapeDtypeStruct((NC, NP, feat), jnp.float32),
        scratch_types=(
            [pltpu.VMEM((CH,), jnp.int32)] * NB
            + [pltpu.VMEM((CH,), jnp.int32)] * NB
            + [pltpu.VMEM((CH, feat), jnp.float32)] * NB
            + [pltpu.VMEM_SHARED((NP, feat), jnp.float32)]
            + [pltpu.SemaphoreType.DMA] * (3 * NB)
        ),
    )
    def agg_kernel(src_hbm, dst_hbm, h_hbm, zeros_hbm, out_hbm, *sc):
        sidx = sc[0:NB]
        didx = sc[NB:2 * NB]
        rows = sc[2 * NB:3 * NB]
        acc = sc[3 * NB]
        gsem = sc[3 * NB + 1:4 * NB + 1]
        dsem = sc[4 * NB + 1:5 * NB + 1]
        ssem = sc[5 * NB + 1:6 * NB + 1]
        c = lax.axis_index("c")
        s = lax.axis_index("s")
        pltpu.sync_copy(zeros_hbm, acc.at[pl.ds(s * RPS, RPS)])
        plsc.subcore_barrier()
        base = (c * NS + s) * EPW

        for b in range(NB):
            off = base + b * CH
            pltpu.sync_copy(src_hbm.at[pl.ds(off, CH)], sidx[b])
            pltpu.async_copy(h_hbm.at[sidx[b]], rows[b], gsem[b])
            pltpu.async_copy(dst_hbm.at[pl.ds(off, CH)], didx[b], dsem[b])

        def round_(j, carry):
            for b in range(NB):
                pltpu.make_async_copy(h_hbm.at[sidx[b]], rows[b],
                                      gsem[b]).wait()
                pltpu.make_async_copy(dst_hbm.at[pl.ds(base, CH)],
                                      didx[b], dsem[b]).wait()
                pltpu.async_copy(rows[b], acc.at[didx[b]], ssem[b], add=True)

            @pl.when(j < NR - 1)
            def _():
                for b in range(NB):
                    offn = base + ((j + 1) * NB + b) * CH
                    pltpu.sync_copy(src_hbm.at[pl.ds(offn, CH)], sidx[b])
                    pltpu.make_async_copy(rows[b], acc.at[didx[b]],
                                          ssem[b]).wait()
                    pltpu.async_copy(h_hbm.at[sidx[b]], rows[b], gsem[b])
                    pltpu.async_copy(dst_hbm.at[pl.ds(offn, CH)],
                                     didx[b], dsem[b])
            return carry

        lax.fori_loop(0, NR, round_, 0)
        for b in range(NB):
            pltpu.make_async_copy(rows[b], acc.at[didx[b]], ssem[b]).wait()
        plsc.subcore_barrier()
        pltpu.sync_copy(acc.at[pl.ds(s * RPS, RPS)],
                        out_hbm.at[c, pl.ds(s * RPS, RPS)])

    return agg_kernel(src, dst, hhat, zeros)


BN = 2000


def _tc_first(degp, x, w):
    def body(degp_ref, x_ref, w_ref, dis_ref, h_ref):
        deg = degp_ref[0, :, 0:1] + degp_ref[1, :, 0:1]
        dis = lax.rsqrt(deg + 1.0)
        dis_ref[...] = dis
        h = jnp.dot(x_ref[...], w_ref[...], preferred_element_type=jnp.float32)
        h_ref[...] = h * dis

    return pl.pallas_call(
        body,
        grid=(N // BN,),
        in_specs=[
            pl.BlockSpec((NC, BN, 128), lambda i: (0, i, 0)),
            pl.BlockSpec((BN, D), lambda i: (i, 0)),
            pl.BlockSpec(w.shape, lambda i: (0, 0)),
        ],
        out_specs=[
            pl.BlockSpec((BN, 1), lambda i: (i, 0)),
            pl.BlockSpec((BN, w.shape[1]), lambda i: (i, 0)),
        ],
        out_shape=[
            jax.ShapeDtypeStruct((N, 1), jnp.float32),
            jax.ShapeDtypeStruct((N, w.shape[1]), jnp.float32),
        ],
    )(degp, x, w)


def _tc_mid(aggp, hhat, dis, b, w):
    f_in = hhat.shape[1]
    f_out = w.shape[1]

    def body(aggp_ref, h_ref, dis_ref, b_ref, w_ref, o_ref):
        t = (aggp_ref[0] + aggp_ref[1] + h_ref[...]) * dis_ref[...] + b_ref[...]
        t = jnp.maximum(t, 0.0)
        o_ref[...] = jnp.dot(t, w_ref[...],
                             preferred_element_type=jnp.float32) * dis_ref[...]

    return pl.pallas_call(
        body,
        grid=(N // BN,),
        in_specs=[
            pl.BlockSpec((NC, BN, f_in), lambda i: (0, i, 0)),
            pl.BlockSpec((BN, f_in), lambda i: (i, 0)),
            pl.BlockSpec((BN, 1), lambda i: (i, 0)),
            pl.BlockSpec((1, f_in), lambda i: (0, 0)),
            pl.BlockSpec((f_in, f_out), lambda i: (0, 0)),
        ],
        out_specs=pl.BlockSpec((BN, f_out), lambda i: (i, 0)),
        out_shape=jax.ShapeDtypeStruct((N, f_out), jnp.float32),
    )(aggp, hhat, dis, b, w)


def _tc_final(aggp, hhat, dis, b):
    f = hhat.shape[1]

    def body(aggp_ref, h_ref, dis_ref, b_ref, o_ref):
        o_ref[...] = ((aggp_ref[0] + aggp_ref[1] + h_ref[...])
                      * dis_ref[...] + b_ref[...])

    return pl.pallas_call(
        body,
        grid=(N // BN,),
        in_specs=[
            pl.BlockSpec((NC, BN, f), lambda i: (0, i, 0)),
            pl.BlockSpec((BN, f), lambda i: (i, 0)),
            pl.BlockSpec((BN, 1), lambda i: (i, 0)),
            pl.BlockSpec((1, f), lambda i: (0, 0)),
        ],
        out_specs=pl.BlockSpec((BN, f), lambda i: (i, 0)),
        out_shape=jax.ShapeDtypeStruct((N, f), jnp.float32),
    )(aggp, hhat, dis, b)


def kernel(x, edge_index, W1, b1, Wm, bm, W2, b2):
    src = edge_index[0]
    dst = edge_index[1]
    H = W1.shape[1]
    C = W2.shape[1]
    CP = 128

    zerosH = jnp.zeros((RPS, H), jnp.float32)
    ones128 = jnp.ones((CH, 128), jnp.float32)
    w2p = jnp.zeros((H, CP), jnp.float32).at[:, :C].set(W2)
    b2p = jnp.zeros((CP,), jnp.float32).at[:C].set(b2)

    degp = _sc_degree(dst, zerosH, ones128)
    dis, h1 = _tc_first(degp, x, W1)

    a1 = _sc_aggregate(src, dst, h1, zerosH, H)
    h2 = _tc_mid(a1, h1, dis, b1.reshape(1, H), Wm)

    a2 = _sc_aggregate(src, dst, h2, zerosH, H)
    h3 = _tc_mid(a2, h2, dis, bm.reshape(1, H), w2p)

    a3 = _sc_aggregate(src, dst, h3, zerosH, CP)
    outp = _tc_final(a3, h3, dis, b2p.reshape(1, CP))
    return outp[:, :C]


# speedup vs baseline: 25.0724x; 1.3188x over previous
import functools

import jax
import jax.numpy as jnp
from jax import lax
from jax.experimental import pallas as pl
from jax.experimental.pallas import tpu as pltpu
from jax.experimental.pallas import tpu_sc as plsc

N = 10000
E = 320000
D = 128

NC = 2
NS = 16
NW = NC * NS
EPW = E // NW
CH = 40
NCH = EPW // CH
NP = 10240
RPS = NP // NS
NB = 5
NR = NCH // NB

_MESH = plsc.VectorSubcoreMesh(core_axis_name="c", subcore_axis_name="s")


def _sc_degree(dst, zeros, ones):
    @functools.partial(
        pl.kernel,
        mesh=_MESH,
        out_type=jax.ShapeDtypeStruct((NC, NP, 128), jnp.float32),
        scratch_types=(
            [pltpu.VMEM((CH,), jnp.int32)] * NB
            + [pltpu.VMEM((CH, 128), jnp.float32),
               pltpu.VMEM_SHARED((NP, 128), jnp.float32)]
            + [pltpu.SemaphoreType.DMA] * (2 * NB)
        ),
    )
    def deg_kernel(dst_hbm, zeros_hbm, ones_hbm, out_hbm, *sc):
        didx = sc[0:NB]
        ones_v = sc[NB]
        acc = sc[NB + 1]
        dsem = sc[NB + 2:2 * NB + 2]
        ssem = sc[2 * NB + 2:3 * NB + 2]
        c = lax.axis_index("c")
        s = lax.axis_index("s")
        pltpu.sync_copy(ones_hbm, ones_v)
        pltpu.sync_copy(zeros_hbm, acc.at[pl.ds(s * RPS, RPS)])
        plsc.subcore_barrier()
        base = (c * NS + s) * EPW

        for b in range(NB):
            pltpu.async_copy(dst_hbm.at[pl.ds(base + b * CH, CH)],
                             didx[b], dsem[b])

        def round_(j, carry):
            for b in range(NB):
                pltpu.make_async_copy(dst_hbm.at[pl.ds(base, CH)],
                                      didx[b], dsem[b]).wait()
                pltpu.async_copy(ones_v, acc.at[didx[b]], ssem[b], add=True)

            @pl.when(j < NR - 1)
            def _():
                for b in range(NB):
                    offn = base + ((j + 1) * NB + b) * CH
                    pltpu.make_async_copy(ones_v, acc.at[didx[b]],
                                          ssem[b]).wait()
                    pltpu.async_copy(dst_hbm.at[pl.ds(offn, CH)],
                                     didx[b], dsem[b])
            return carry

        lax.fori_loop(0, NR, round_, 0)
        for b in range(NB):
            pltpu.make_async_copy(ones_v, acc.at[didx[b]], ssem[b]).wait()
        plsc.subcore_barrier()
        pltpu.sync_copy(acc.at[pl.ds(s * RPS, RPS)],
                        out_hbm.at[c, pl.ds(s * RPS, RPS)])

    return deg_kernel(dst, zeros, ones)


def _sc_aggregate(src, dst, hhat, zeros, feat):
    @functools.partial(
        pl.kernel,
        mesh=_MESH,
        out_type=jax.ShapeDtypeStruct((NC, NP, feat), jnp.float32),
        scratch_types=(
            [pltpu.VMEM((CH,), jnp.int32)] * NB
            + [pltpu.VMEM((CH,), jnp.int32)] * NB
            + [pltpu.VMEM((CH, feat), jnp.float32)] * NB
            + [pltpu.VMEM_SHARED((NP, feat), jnp.float32)]
            + [pltpu.SemaphoreType.DMA] * (4 * NB)
        ),
    )
    def agg_kernel(src_hbm, dst_hbm, h_hbm, zeros_hbm, out_hbm, *sc):
        sidx = sc[0:NB]
        didx = sc[NB:2 * NB]
        rows = sc[2 * NB:3 * NB]
        acc = sc[3 * NB]
        gsem = sc[3 * NB + 1:4 * NB + 1]
        dsem = sc[4 * NB + 1:5 * NB + 1]
        ssem = sc[5 * NB + 1:6 * NB + 1]
        isem = sc[6 * NB + 1:7 * NB + 1]
        c = lax.axis_index("c")
        s = lax.axis_index("s")
        pltpu.sync_copy(zeros_hbm, acc.at[pl.ds(s * RPS, RPS)])
        plsc.subcore_barrier()
        base = (c * NS + s) * EPW

        for b in range(NB):
            off = base + b * CH
            pltpu.sync_copy(src_hbm.at[pl.ds(off, CH)], sidx[b])
            pltpu.async_copy(h_hbm.at[sidx[b]], rows[b], gsem[b])
            pltpu.async_copy(dst_hbm.at[pl.ds(off, CH)], didx[b], dsem[b])

        def round_(j, carry):
            for b in range(NB):
                pltpu.make_async_copy(h_hbm.at[sidx[b]], rows[b],
                                      gsem[b]).wait()

                @pl.when(j < NR - 1)
                def _():
                    offn = base + ((j + 1) * NB + b) * CH
                    pltpu.async_copy(src_hbm.at[pl.ds(offn, CH)],
                                     sidx[b], isem[b])

                pltpu.make_async_copy(dst_hbm.at[pl.ds(base, CH)],
                                      didx[b], dsem[b]).wait()
                pltpu.async_copy(rows[b], acc.at[didx[b]], ssem[b], add=True)

            @pl.when(j < NR - 1)
            def _():
                for b in range(NB):
                    offn = base + ((j + 1) * NB + b) * CH
                    pltpu.make_async_copy(src_hbm.at[pl.ds(offn, CH)],
                                          sidx[b], isem[b]).wait()
                    pltpu.make_async_copy(rows[b], acc.at[didx[b]],
                                          ssem[b]).wait()
                    pltpu.async_copy(h_hbm.at[sidx[b]], rows[b], gsem[b])
                    pltpu.async_copy(dst_hbm.at[pl.ds(offn, CH)],
                                     didx[b], dsem[b])
            return carry

        lax.fori_loop(0, NR, round_, 0)
        for b in range(NB):
            pltpu.make_async_copy(rows[b], acc.at[didx[b]], ssem[b]).wait()
        plsc.subcore_barrier()
        pltpu.sync_copy(acc.at[pl.ds(s * RPS, RPS)],
                        out_hbm.at[c, pl.ds(s * RPS, RPS)])

    return agg_kernel(src, dst, hhat, zeros)


BN = 2000


def _tc_first(degp, x, w):
    def body(degp_ref, x_ref, w_ref, dis_ref, h_ref):
        deg = degp_ref[0, :, 0:1] + degp_ref[1, :, 0:1]
        dis = lax.rsqrt(deg + 1.0)
        dis_ref[...] = dis
        h = jnp.dot(x_ref[...], w_ref[...], preferred_element_type=jnp.float32)
        h_ref[...] = h * dis

    return pl.pallas_call(
        body,
        grid=(N // BN,),
        in_specs=[
            pl.BlockSpec((NC, BN, 128), lambda i: (0, i, 0)),
            pl.BlockSpec((BN, D), lambda i: (i, 0)),
            pl.BlockSpec(w.shape, lambda i: (0, 0)),
        ],
        out_specs=[
            pl.BlockSpec((BN, 1), lambda i: (i, 0)),
            pl.BlockSpec((BN, w.shape[1]), lambda i: (i, 0)),
        ],
        out_shape=[
            jax.ShapeDtypeStruct((N, 1), jnp.float32),
            jax.ShapeDtypeStruct((N, w.shape[1]), jnp.float32),
        ],
    )(degp, x, w)


def _tc_mid(aggp, hhat, dis, b, w):
    f_in = hhat.shape[1]
    f_out = w.shape[1]

    def body(aggp_ref, h_ref, dis_ref, b_ref, w_ref, o_ref):
        t = (aggp_ref[0] + aggp_ref[1] + h_ref[...]) * dis_ref[...] + b_ref[...]
        t = jnp.maximum(t, 0.0)
        o_ref[...] = jnp.dot(t, w_ref[...],
                             preferred_element_type=jnp.float32) * dis_ref[...]

    return pl.pallas_call(
        body,
        grid=(N // BN,),
        in_specs=[
            pl.BlockSpec((NC, BN, f_in), lambda i: (0, i, 0)),
            pl.BlockSpec((BN, f_in), lambda i: (i, 0)),
            pl.BlockSpec((BN, 1), lambda i: (i, 0)),
            pl.BlockSpec((1, f_in), lambda i: (0, 0)),
            pl.BlockSpec((f_in, f_out), lambda i: (0, 0)),
        ],
        out_specs=pl.BlockSpec((BN, f_out), lambda i: (i, 0)),
        out_shape=jax.ShapeDtypeStruct((N, f_out), jnp.float32),
    )(aggp, hhat, dis, b, w)


def _tc_final(aggp, hhat, dis, b):
    f = hhat.shape[1]

    def body(aggp_ref, h_ref, dis_ref, b_ref, o_ref):
        o_ref[...] = ((aggp_ref[0] + aggp_ref[1] + h_ref[...])
                      * dis_ref[...] + b_ref[...])

    return pl.pallas_call(
        body,
        grid=(N // BN,),
        in_specs=[
            pl.BlockSpec((NC, BN, f), lambda i: (0, i, 0)),
            pl.BlockSpec((BN, f), lambda i: (i, 0)),
            pl.BlockSpec((BN, 1), lambda i: (i, 0)),
            pl.BlockSpec((1, f), lambda i: (0, 0)),
        ],
        out_specs=pl.BlockSpec((BN, f), lambda i: (i, 0)),
        out_shape=jax.ShapeDtypeStruct((N, f), jnp.float32),
    )(aggp, hhat, dis, b)


def kernel(x, edge_index, W1, b1, Wm, bm, W2, b2):
    src = edge_index[0]
    dst = edge_index[1]
    H = W1.shape[1]
    C = W2.shape[1]
    CP = 128

    zerosH = jnp.zeros((RPS, H), jnp.float32)
    ones128 = jnp.ones((CH, 128), jnp.float32)
    w2p = jnp.zeros((H, CP), jnp.float32).at[:, :C].set(W2)
    b2p = jnp.zeros((CP,), jnp.float32).at[:C].set(b2)

    degp = _sc_degree(dst, zerosH, ones128)
    dis, h1 = _tc_first(degp, x, W1)

    a1 = _sc_aggregate(src, dst, h1, zerosH, H)
    h2 = _tc_mid(a1, h1, dis, b1.reshape(1, H), Wm)

    a2 = _sc_aggregate(src, dst, h2, zerosH, H)
    h3 = _tc_mid(a2, h2, dis, bm.reshape(1, H), w2p)

    a3 = _sc_aggregate(src, dst, h3, zerosH, CP)
    outp = _tc_final(a3, h3, dis, b2p.reshape(1, CP))
    return outp[:, :C]


# speedup vs baseline: 25.3314x; 1.0103x over previous
import functools

import jax
import jax.numpy as jnp
from jax import lax
from jax.experimental import pallas as pl
from jax.experimental.pallas import tpu as pltpu
from jax.experimental.pallas import tpu_sc as plsc

N = 10000
E = 320000
D = 128

NC = 2
NS = 16
NW = NC * NS
EPW = E // NW
CH = 80
NCH = EPW // CH
NP = 10240
RPS = NP // NS
NBD = 5
NRD = NCH // NBD
NBA = 4
NRA = NCH // NBA

_MESH = plsc.VectorSubcoreMesh(core_axis_name="c", subcore_axis_name="s")


def _sc_degree(dst, zeros, ones):
    @functools.partial(
        pl.kernel,
        mesh=_MESH,
        out_type=jax.ShapeDtypeStruct((NC, NP, 128), jnp.float32),
        scratch_types=(
            [pltpu.VMEM((CH,), jnp.int32)] * NBD
            + [pltpu.VMEM((CH, 128), jnp.float32),
               pltpu.VMEM_SHARED((NP, 128), jnp.float32)]
            + [pltpu.SemaphoreType.DMA] * (2 * NBD)
        ),
    )
    def deg_kernel(dst_hbm, zeros_hbm, ones_hbm, out_hbm, *sc):
        didx = sc[0:NBD]
        ones_v = sc[NBD]
        acc = sc[NBD + 1]
        dsem = sc[NBD + 2:2 * NBD + 2]
        ssem = sc[2 * NBD + 2:3 * NBD + 2]
        c = lax.axis_index("c")
        s = lax.axis_index("s")
        pltpu.sync_copy(ones_hbm, ones_v)
        pltpu.sync_copy(zeros_hbm, acc.at[pl.ds(s * RPS, RPS)])
        plsc.subcore_barrier()
        base = (c * NS + s) * EPW

        for b in range(NBD):
            pltpu.async_copy(dst_hbm.at[pl.ds(base + b * CH, CH)],
                             didx[b], dsem[b])

        def round_(j, carry):
            for b in range(NBD):
                pltpu.make_async_copy(dst_hbm.at[pl.ds(base, CH)],
                                      didx[b], dsem[b]).wait()
                pltpu.async_copy(ones_v, acc.at[didx[b]], ssem[b], add=True)

            @pl.when(j < NRD - 1)
            def _():
                for b in range(NBD):
                    offn = base + ((j + 1) * NBD + b) * CH
                    pltpu.make_async_copy(ones_v, acc.at[didx[b]],
                                          ssem[b]).wait()
                    pltpu.async_copy(dst_hbm.at[pl.ds(offn, CH)],
                                     didx[b], dsem[b])
            return carry

        lax.fori_loop(0, NRD, round_, 0)
        for b in range(NBD):
            pltpu.make_async_copy(ones_v, acc.at[didx[b]], ssem[b]).wait()
        plsc.subcore_barrier()
        pltpu.sync_copy(acc.at[pl.ds(s * RPS, RPS)],
                        out_hbm.at[c, pl.ds(s * RPS, RPS)])

    return deg_kernel(dst, zeros, ones)


def _sc_aggregate(src, dst, hhat, zeros, feat):
    @functools.partial(
        pl.kernel,
        mesh=_MESH,
        out_type=jax.ShapeDtypeStruct((NC, NP, feat), jnp.float32),
        scratch_types=(
            [pltpu.VMEM((CH,), jnp.int32)] * NBA
            + [pltpu.VMEM((CH,), jnp.int32)] * NBA
            + [pltpu.VMEM((CH, feat), jnp.float32)] * NBA
            + [pltpu.VMEM_SHARED((NP, feat), jnp.float32)]
            + [pltpu.SemaphoreType.DMA] * (4 * NBA)
        ),
    )
    def agg_kernel(src_hbm, dst_hbm, h_hbm, zeros_hbm, out_hbm, *sc):
        sidx = sc[0:NBA]
        didx = sc[NBA:2 * NBA]
        rows = sc[2 * NBA:3 * NBA]
        acc = sc[3 * NBA]
        gsem = sc[3 * NBA + 1:4 * NBA + 1]
        dsem = sc[4 * NBA + 1:5 * NBA + 1]
        ssem = sc[5 * NBA + 1:6 * NBA + 1]
        isem = sc[6 * NBA + 1:7 * NBA + 1]
        c = lax.axis_index("c")
        s = lax.axis_index("s")
        pltpu.sync_copy(zeros_hbm, acc.at[pl.ds(s * RPS, RPS)])
        plsc.subcore_barrier()
        base = (c * NS + s) * EPW

        for b in range(NBA):
            off = base + b * CH
            pltpu.sync_copy(src_hbm.at[pl.ds(off, CH)], sidx[b])
            pltpu.async_copy(h_hbm.at[sidx[b]], rows[b], gsem[b])
            pltpu.async_copy(dst_hbm.at[pl.ds(off, CH)], didx[b], dsem[b])

        def round_(j, carry):
            for b in range(NBA):
                pltpu.make_async_copy(h_hbm.at[sidx[b]], rows[b],
                                      gsem[b]).wait()

                @pl.when(j < NRA - 1)
                def _():
                    offn = base + ((j + 1) * NBA + b) * CH
                    pltpu.async_copy(src_hbm.at[pl.ds(offn, CH)],
                                     sidx[b], isem[b])

                pltpu.make_async_copy(dst_hbm.at[pl.ds(base, CH)],
                                      didx[b], dsem[b]).wait()
                pltpu.async_copy(rows[b], acc.at[didx[b]], ssem[b], add=True)

            @pl.when(j < NRA - 1)
            def _():
                for b in range(NBA):
                    offn = base + ((j + 1) * NBA + b) * CH
                    pltpu.make_async_copy(src_hbm.at[pl.ds(offn, CH)],
                                          sidx[b], isem[b]).wait()
                    pltpu.make_async_copy(rows[b], acc.at[didx[b]],
                                          ssem[b]).wait()
                    pltpu.async_copy(h_hbm.at[sidx[b]], rows[b], gsem[b])
                    pltpu.async_copy(dst_hbm.at[pl.ds(offn, CH)],
                                     didx[b], dsem[b])
            return carry

        lax.fori_loop(0, NRA, round_, 0)
        offt = base + NBA * NRA * CH
        pltpu.sync_copy(src_hbm.at[pl.ds(offt, CH)], sidx[0])
        pltpu.make_async_copy(rows[0], acc.at[didx[0]], ssem[0]).wait()
        pltpu.async_copy(h_hbm.at[sidx[0]], rows[0], gsem[0]).wait()
        pltpu.sync_copy(dst_hbm.at[pl.ds(offt, CH)], didx[0])
        pltpu.async_copy(rows[0], acc.at[didx[0]], ssem[0], add=True)
        for b in range(NBA):
            pltpu.make_async_copy(rows[b], acc.at[didx[b]], ssem[b]).wait()
        plsc.subcore_barrier()
        pltpu.sync_copy(acc.at[pl.ds(s * RPS, RPS)],
                        out_hbm.at[c, pl.ds(s * RPS, RPS)])

    return agg_kernel(src, dst, hhat, zeros)


BN = 2000


def _tc_first(degp, x, w):
    def body(degp_ref, x_ref, w_ref, dis_ref, h_ref):
        deg = degp_ref[0, :, 0:1] + degp_ref[1, :, 0:1]
        dis = lax.rsqrt(deg + 1.0)
        dis_ref[...] = dis
        h = jnp.dot(x_ref[...], w_ref[...], preferred_element_type=jnp.float32)
        h_ref[...] = h * dis

    return pl.pallas_call(
        body,
        grid=(N // BN,),
        in_specs=[
            pl.BlockSpec((NC, BN, 128), lambda i: (0, i, 0)),
            pl.BlockSpec((BN, D), lambda i: (i, 0)),
            pl.BlockSpec(w.shape, lambda i: (0, 0)),
        ],
        out_specs=[
            pl.BlockSpec((BN, 1), lambda i: (i, 0)),
            pl.BlockSpec((BN, w.shape[1]), lambda i: (i, 0)),
        ],
        out_shape=[
            jax.ShapeDtypeStruct((N, 1), jnp.float32),
            jax.ShapeDtypeStruct((N, w.shape[1]), jnp.float32),
        ],
    )(degp, x, w)


def _tc_mid(aggp, hhat, dis, b, w):
    f_in = hhat.shape[1]
    f_out = w.shape[1]

    def body(aggp_ref, h_ref, dis_ref, b_ref, w_ref, o_ref):
        t = (aggp_ref[0] + aggp_ref[1] + h_ref[...]) * dis_ref[...] + b_ref[...]
        t = jnp.maximum(t, 0.0)
        o_ref[...] = jnp.dot(t, w_ref[...],
                             preferred_element_type=jnp.float32) * dis_ref[...]

    return pl.pallas_call(
        body,
        grid=(N // BN,),
        in_specs=[
            pl.BlockSpec((NC, BN, f_in), lambda i: (0, i, 0)),
            pl.BlockSpec((BN, f_in), lambda i: (i, 0)),
            pl.BlockSpec((BN, 1), lambda i: (i, 0)),
            pl.BlockSpec((1, f_in), lambda i: (0, 0)),
            pl.BlockSpec((f_in, f_out), lambda i: (0, 0)),
        ],
        out_specs=pl.BlockSpec((BN, f_out), lambda i: (i, 0)),
        out_shape=jax.ShapeDtypeStruct((N, f_out), jnp.float32),
    )(aggp, hhat, dis, b, w)


def _tc_final(aggp, hhat, dis, b):
    f = hhat.shape[1]

    def body(aggp_ref, h_ref, dis_ref, b_ref, o_ref):
        o_ref[...] = ((aggp_ref[0] + aggp_ref[1] + h_ref[...])
                      * dis_ref[...] + b_ref[...])

    return pl.pallas_call(
        body,
        grid=(N // BN,),
        in_specs=[
            pl.BlockSpec((NC, BN, f), lambda i: (0, i, 0)),
            pl.BlockSpec((BN, f), lambda i: (i, 0)),
            pl.BlockSpec((BN, 1), lambda i: (i, 0)),
            pl.BlockSpec((1, f), lambda i: (0, 0)),
        ],
        out_specs=pl.BlockSpec((BN, f), lambda i: (i, 0)),
        out_shape=jax.ShapeDtypeStruct((N, f), jnp.float32),
    )(aggp, hhat, dis, b)


def kernel(x, edge_index, W1, b1, Wm, bm, W2, b2):
    src = edge_index[0]
    dst = edge_index[1]
    H = W1.shape[1]
    C = W2.shape[1]
    CP = 128

    zerosH = jnp.zeros((RPS, H), jnp.float32)
    ones128 = jnp.ones((CH, 128), jnp.float32)
    w2p = jnp.zeros((H, CP), jnp.float32).at[:, :C].set(W2)
    b2p = jnp.zeros((CP,), jnp.float32).at[:C].set(b2)

    degp = _sc_degree(dst, zerosH, ones128)
    dis, h1 = _tc_first(degp, x, W1)

    a1 = _sc_aggregate(src, dst, h1, zerosH, H)
    h2 = _tc_mid(a1, h1, dis, b1.reshape(1, H), Wm)

    a2 = _sc_aggregate(src, dst, h2, zerosH, H)
    h3 = _tc_mid(a2, h2, dis, bm.reshape(1, H), w2p)

    a3 = _sc_aggregate(src, dst, h3, zerosH, CP)
    outp = _tc_final(a3, h3, dis, b2p.reshape(1, CP))
    return outp[:, :C]


# speedup vs baseline: 25.6788x; 1.0137x over previous
import functools

import jax
import jax.numpy as jnp
from jax import lax
from jax.experimental import pallas as pl
from jax.experimental.pallas import tpu as pltpu
from jax.experimental.pallas import tpu_sc as plsc

N = 10000
E = 320000
D = 128

NC = 2
NS = 16
NW = NC * NS
EPW = E // NW
CH = 80
NCH = EPW // CH
NP = 10240
RPS = NP // NS
NBD = 5
NRD = NCH // NBD
NBA = 4
NRA = NCH // NBA

_MESH = plsc.VectorSubcoreMesh(core_axis_name="c", subcore_axis_name="s")


def _sc_degree(dst, zeros, ones):
    @functools.partial(
        pl.kernel,
        mesh=_MESH,
        out_type=jax.ShapeDtypeStruct((NC, NP, 128), jnp.float32),
        scratch_types=(
            [pltpu.VMEM((CH,), jnp.int32)] * NBD
            + [pltpu.VMEM((CH, 128), jnp.float32),
               pltpu.VMEM_SHARED((NP, 128), jnp.float32)]
            + [pltpu.SemaphoreType.DMA] * (2 * NBD)
        ),
    )
    def deg_kernel(dst_hbm, zeros_hbm, ones_hbm, out_hbm, *sc):
        didx = sc[0:NBD]
        ones_v = sc[NBD]
        acc = sc[NBD + 1]
        dsem = sc[NBD + 2:2 * NBD + 2]
        ssem = sc[2 * NBD + 2:3 * NBD + 2]
        c = lax.axis_index("c")
        s = lax.axis_index("s")
        pltpu.sync_copy(ones_hbm, ones_v)
        base = (c * NS + s) * EPW
        for b in range(NBD):
            pltpu.async_copy(dst_hbm.at[pl.ds(base + b * CH, CH)],
                             didx[b], dsem[b])
        pltpu.sync_copy(zeros_hbm.at[pl.ds(s * RPS, RPS)],
                        acc.at[pl.ds(s * RPS, RPS)])
        plsc.subcore_barrier()

        def round_(j, carry):
            for b in range(NBD):
                pltpu.make_async_copy(dst_hbm.at[pl.ds(base, CH)],
                                      didx[b], dsem[b]).wait()
                pltpu.async_copy(ones_v, acc.at[didx[b]], ssem[b], add=True)

            @pl.when(j < NRD - 1)
            def _():
                for b in range(NBD):
                    offn = base + ((j + 1) * NBD + b) * CH
                    pltpu.make_async_copy(ones_v, acc.at[didx[b]],
                                          ssem[b]).wait()
                    pltpu.async_copy(dst_hbm.at[pl.ds(offn, CH)],
                                     didx[b], dsem[b])
            return carry

        lax.fori_loop(0, NRD, round_, 0)
        for b in range(NBD):
            pltpu.make_async_copy(ones_v, acc.at[didx[b]], ssem[b]).wait()
        plsc.subcore_barrier()
        pltpu.sync_copy(acc.at[pl.ds(s * RPS, RPS)],
                        out_hbm.at[c, pl.ds(s * RPS, RPS)])

    return deg_kernel(dst, zeros, ones)


def _sc_aggregate(src, dst, hhat, zeros, feat):
    @functools.partial(
        pl.kernel,
        mesh=_MESH,
        out_type=jax.ShapeDtypeStruct((NC, NP, feat), jnp.float32),
        scratch_types=(
            [pltpu.VMEM((CH,), jnp.int32)] * NBA
            + [pltpu.VMEM((CH,), jnp.int32)] * NBA
            + [pltpu.VMEM((CH, feat), jnp.float32)] * NBA
            + [pltpu.VMEM_SHARED((NP, feat), jnp.float32)]
            + [pltpu.SemaphoreType.DMA] * (4 * NBA)
        ),
    )
    def agg_kernel(src_hbm, dst_hbm, h_hbm, zeros_hbm, out_hbm, *sc):
        sidx = sc[0:NBA]
        didx = sc[NBA:2 * NBA]
        rows = sc[2 * NBA:3 * NBA]
        acc = sc[3 * NBA]
        gsem = sc[3 * NBA + 1:4 * NBA + 1]
        dsem = sc[4 * NBA + 1:5 * NBA + 1]
        ssem = sc[5 * NBA + 1:6 * NBA + 1]
        isem = sc[6 * NBA + 1:7 * NBA + 1]
        c = lax.axis_index("c")
        s = lax.axis_index("s")
        base = (c * NS + s) * EPW
        for b in range(NBA):
            off = base + b * CH
            pltpu.sync_copy(src_hbm.at[pl.ds(off, CH)], sidx[b])
            pltpu.async_copy(h_hbm.at[sidx[b]], rows[b], gsem[b])
            pltpu.async_copy(dst_hbm.at[pl.ds(off, CH)], didx[b], dsem[b])
        pltpu.sync_copy(zeros_hbm.at[pl.ds(s * RPS, RPS)],
                        acc.at[pl.ds(s * RPS, RPS)])
        plsc.subcore_barrier()

        def round_(j, carry):
            for b in range(NBA):
                pltpu.make_async_copy(h_hbm.at[sidx[b]], rows[b],
                                      gsem[b]).wait()

                @pl.when(j < NRA - 1)
                def _():
                    offn = base + ((j + 1) * NBA + b) * CH
                    pltpu.async_copy(src_hbm.at[pl.ds(offn, CH)],
                                     sidx[b], isem[b])

                pltpu.make_async_copy(dst_hbm.at[pl.ds(base, CH)],
                                      didx[b], dsem[b]).wait()
                pltpu.async_copy(rows[b], acc.at[didx[b]], ssem[b], add=True)

            @pl.when(j < NRA - 1)
            def _():
                for b in range(NBA):
                    offn = base + ((j + 1) * NBA + b) * CH
                    pltpu.make_async_copy(src_hbm.at[pl.ds(offn, CH)],
                                          sidx[b], isem[b]).wait()
                    pltpu.make_async_copy(rows[b], acc.at[didx[b]],
                                          ssem[b]).wait()
                    pltpu.async_copy(h_hbm.at[sidx[b]], rows[b], gsem[b])
                    pltpu.async_copy(dst_hbm.at[pl.ds(offn, CH)],
                                     didx[b], dsem[b])
            return carry

        lax.fori_loop(0, NRA, round_, 0)
        offt = base + NBA * NRA * CH
        pltpu.sync_copy(src_hbm.at[pl.ds(offt, CH)], sidx[0])
        pltpu.make_async_copy(rows[0], acc.at[didx[0]], ssem[0]).wait()
        pltpu.async_copy(h_hbm.at[sidx[0]], rows[0], gsem[0]).wait()
        pltpu.sync_copy(dst_hbm.at[pl.ds(offt, CH)], didx[0])
        pltpu.async_copy(rows[0], acc.at[didx[0]], ssem[0], add=True)
        for b in range(NBA):
            pltpu.make_async_copy(rows[b], acc.at[didx[b]], ssem[b]).wait()
        plsc.subcore_barrier()
        pltpu.sync_copy(acc.at[pl.ds(s * RPS, RPS)],
                        out_hbm.at[c, pl.ds(s * RPS, RPS)])

    return agg_kernel(src, dst, hhat, zeros)


BN = 2000


def _tc_first(degp, x, w):
    def body(degp_ref, x_ref, w_ref, dis_ref, h_ref):
        deg = degp_ref[0, :, 0:1] + degp_ref[1, :, 0:1]
        dis = lax.rsqrt(deg + 1.0)
        dis_ref[...] = dis
        h = jnp.dot(x_ref[...], w_ref[...], preferred_element_type=jnp.float32)
        h_ref[...] = h * dis

    return pl.pallas_call(
        body,
        grid=(N // BN,),
        in_specs=[
            pl.BlockSpec((NC, BN, 128), lambda i: (0, i, 0)),
            pl.BlockSpec((BN, D), lambda i: (i, 0)),
            pl.BlockSpec(w.shape, lambda i: (0, 0)),
        ],
        out_specs=[
            pl.BlockSpec((BN, 1), lambda i: (i, 0)),
            pl.BlockSpec((BN, w.shape[1]), lambda i: (i, 0)),
        ],
        out_shape=[
            jax.ShapeDtypeStruct((N, 1), jnp.float32),
            jax.ShapeDtypeStruct((N, w.shape[1]), jnp.float32),
        ],
    )(degp, x, w)


def _tc_mid(aggp, hhat, dis, b, w):
    f_in = hhat.shape[1]
    f_out = w.shape[1]

    def body(aggp_ref, h_ref, dis_ref, b_ref, w_ref, o_ref):
        t = (aggp_ref[0] + aggp_ref[1] + h_ref[...]) * dis_ref[...] + b_ref[...]
        t = jnp.maximum(t, 0.0)
        o_ref[...] = jnp.dot(t, w_ref[...],
                             preferred_element_type=jnp.float32) * dis_ref[...]

    return pl.pallas_call(
        body,
        grid=(N // BN,),
        in_specs=[
            pl.BlockSpec((NC, BN, f_in), lambda i: (0, i, 0)),
            pl.BlockSpec((BN, f_in), lambda i: (i, 0)),
            pl.BlockSpec((BN, 1), lambda i: (i, 0)),
            pl.BlockSpec((1, f_in), lambda i: (0, 0)),
            pl.BlockSpec((f_in, f_out), lambda i: (0, 0)),
        ],
        out_specs=pl.BlockSpec((BN, f_out), lambda i: (i, 0)),
        out_shape=jax.ShapeDtypeStruct((N, f_out), jnp.float32),
    )(aggp, hhat, dis, b, w)


def _tc_final(aggp, hhat, dis, b):
    f = hhat.shape[1]

    def body(aggp_ref, h_ref, dis_ref, b_ref, o_ref):
        o_ref[...] = ((aggp_ref[0] + aggp_ref[1] + h_ref[...])
                      * dis_ref[...] + b_ref[...])

    return pl.pallas_call(
        body,
        grid=(N // BN,),
        in_specs=[
            pl.BlockSpec((NC, BN, f), lambda i: (0, i, 0)),
            pl.BlockSpec((BN, f), lambda i: (i, 0)),
            pl.BlockSpec((BN, 1), lambda i: (i, 0)),
            pl.BlockSpec((1, f), lambda i: (0, 0)),
        ],
        out_specs=pl.BlockSpec((BN, f), lambda i: (i, 0)),
        out_shape=jax.ShapeDtypeStruct((N, f), jnp.float32),
    )(aggp, hhat, dis, b)


def kernel(x, edge_index, W1, b1, Wm, bm, W2, b2):
    src = edge_index[0]
    dst = edge_index[1]
    H = W1.shape[1]
    C = W2.shape[1]
    CP = 128

    zerosH = jnp.zeros((NP, H), jnp.float32)
    ones128 = jnp.ones((CH, 128), jnp.float32)
    w2p = jnp.zeros((H, CP), jnp.float32).at[:, :C].set(W2)
    b2p = jnp.zeros((CP,), jnp.float32).at[:C].set(b2)

    degp = _sc_degree(dst, zerosH, ones128)
    dis, h1 = _tc_first(degp, x, W1)

    a1 = _sc_aggregate(src, dst, h1, zerosH, H)
    h2 = _tc_mid(a1, h1, dis, b1.reshape(1, H), Wm)

    a2 = _sc_aggregate(src, dst, h2, zerosH, H)
    h3 = _tc_mid(a2, h2, dis, bm.reshape(1, H), w2p)

    a3 = _sc_aggregate(src, dst, h3, zerosH, CP)
    outp = _tc_final(a3, h3, dis, b2p.reshape(1, CP))
    return outp[:, :C]


# speedup vs baseline: 25.7170x; 1.0015x over previous
import functools

import jax
import jax.numpy as jnp
from jax import lax
from jax.experimental import pallas as pl
from jax.experimental.pallas import tpu as pltpu
from jax.experimental.pallas import tpu_sc as plsc

N = 10000
E = 320000
D = 128

NC = 2
NS = 16
NW = NC * NS
EPW = E // NW
CH = 80
NCH = EPW // CH
NP = 10240
RPS = NP // NS
NBD = 5
NRD = NCH // NBD
CHA = 80
NCHA = EPW // CHA
NBA = 4
NRA = NCHA // NBA
NTA = NCHA - NBA * NRA

_MESH = plsc.VectorSubcoreMesh(core_axis_name="c", subcore_axis_name="s")


def _sc_degree(dst, zeros, ones):
    @functools.partial(
        pl.kernel,
        mesh=_MESH,
        out_type=jax.ShapeDtypeStruct((NC, NP, 128), jnp.float32),
        scratch_types=(
            [pltpu.VMEM((CH,), jnp.int32)] * NBD
            + [pltpu.VMEM((CH, 128), jnp.float32),
               pltpu.VMEM_SHARED((NP, 128), jnp.float32)]
            + [pltpu.SemaphoreType.DMA] * (2 * NBD)
        ),
    )
    def deg_kernel(dst_hbm, zeros_hbm, ones_hbm, out_hbm, *sc):
        didx = sc[0:NBD]
        ones_v = sc[NBD]
        acc = sc[NBD + 1]
        dsem = sc[NBD + 2:2 * NBD + 2]
        ssem = sc[2 * NBD + 2:3 * NBD + 2]
        c = lax.axis_index("c")
        s = lax.axis_index("s")
        pltpu.sync_copy(ones_hbm, ones_v)
        base = (c * NS + s) * EPW
        for b in range(NBD):
            pltpu.async_copy(dst_hbm.at[pl.ds(base + b * CH, CH)],
                             didx[b], dsem[b])
        pltpu.sync_copy(zeros_hbm.at[pl.ds(s * RPS, RPS)],
                        acc.at[pl.ds(s * RPS, RPS)])
        plsc.subcore_barrier()

        def round_(j, carry):
            for b in range(NBD):
                pltpu.make_async_copy(dst_hbm.at[pl.ds(base, CH)],
                                      didx[b], dsem[b]).wait()
                pltpu.async_copy(ones_v, acc.at[didx[b]], ssem[b], add=True)

            @pl.when(j < NRD - 1)
            def _():
                for b in range(NBD):
                    offn = base + ((j + 1) * NBD + b) * CH
                    pltpu.make_async_copy(ones_v, acc.at[didx[b]],
                                          ssem[b]).wait()
                    pltpu.async_copy(dst_hbm.at[pl.ds(offn, CH)],
                                     didx[b], dsem[b])
            return carry

        lax.fori_loop(0, NRD, round_, 0)
        for b in range(NBD):
            pltpu.make_async_copy(ones_v, acc.at[didx[b]], ssem[b]).wait()
        plsc.subcore_barrier()
        pltpu.sync_copy(acc.at[pl.ds(s * RPS, RPS)],
                        out_hbm.at[c, pl.ds(s * RPS, RPS)])

    return deg_kernel(dst, zeros, ones)


def _sc_aggregate(src, dst, hhat, zeros, feat):
    @functools.partial(
        pl.kernel,
        mesh=_MESH,
        out_type=jax.ShapeDtypeStruct((NC, NP, feat), jnp.float32),
        scratch_types=(
            [pltpu.VMEM((CHA,), jnp.int32)] * NBA
            + [pltpu.VMEM((CHA,), jnp.int32)] * NBA
            + [pltpu.VMEM((CHA, feat), jnp.float32)] * NBA
            + [pltpu.VMEM_SHARED((NP, feat), jnp.float32)]
            + [pltpu.SemaphoreType.DMA] * (4 * NBA)
        ),
    )
    def agg_kernel(src_hbm, dst_hbm, h_hbm, zeros_hbm, out_hbm, *sc):
        sidx = sc[0:NBA]
        didx = sc[NBA:2 * NBA]
        rows = sc[2 * NBA:3 * NBA]
        acc = sc[3 * NBA]
        gsem = sc[3 * NBA + 1:4 * NBA + 1]
        dsem = sc[4 * NBA + 1:5 * NBA + 1]
        ssem = sc[5 * NBA + 1:6 * NBA + 1]
        isem = sc[6 * NBA + 1:7 * NBA + 1]
        c = lax.axis_index("c")
        s = lax.axis_index("s")
        base = (c * NS + s) * EPW
        for b in range(NBA):
            off = base + b * CHA
            pltpu.sync_copy(src_hbm.at[pl.ds(off, CHA)], sidx[b])
            pltpu.async_copy(h_hbm.at[sidx[b]], rows[b], gsem[b])
            pltpu.async_copy(dst_hbm.at[pl.ds(off, CHA)], didx[b], dsem[b])
        pltpu.sync_copy(zeros_hbm.at[pl.ds(s * RPS, RPS)],
                        acc.at[pl.ds(s * RPS, RPS)])
        plsc.subcore_barrier()

        def round_(j, carry):
            for b in range(NBA):
                pltpu.make_async_copy(h_hbm.at[sidx[b]], rows[b],
                                      gsem[b]).wait()

                @pl.when(j < NRA - 1)
                def _():
                    offn = base + ((j + 1) * NBA + b) * CHA
                    pltpu.async_copy(src_hbm.at[pl.ds(offn, CHA)],
                                     sidx[b], isem[b])

                pltpu.make_async_copy(dst_hbm.at[pl.ds(base, CHA)],
                                      didx[b], dsem[b]).wait()
                pltpu.async_copy(rows[b], acc.at[didx[b]], ssem[b], add=True)

            @pl.when(j < NRA - 1)
            def _():
                for b in range(NBA):
                    offn = base + ((j + 1) * NBA + b) * CHA
                    pltpu.make_async_copy(src_hbm.at[pl.ds(offn, CHA)],
                                          sidx[b], isem[b]).wait()
                    pltpu.make_async_copy(rows[b], acc.at[didx[b]],
                                          ssem[b]).wait()
                    pltpu.async_copy(h_hbm.at[sidx[b]], rows[b], gsem[b])
                    pltpu.async_copy(dst_hbm.at[pl.ds(offn, CHA)],
                                     didx[b], dsem[b])
            return carry

        lax.fori_loop(0, NRA, round_, 0)
        for t in range(NTA):
            offt = base + (NBA * NRA + t) * CHA
            pltpu.sync_copy(src_hbm.at[pl.ds(offt, CHA)], sidx[t])
            pltpu.make_async_copy(rows[t], acc.at[didx[t]], ssem[t]).wait()
            pltpu.async_copy(h_hbm.at[sidx[t]], rows[t], gsem[t]).wait()
            pltpu.sync_copy(dst_hbm.at[pl.ds(offt, CHA)], didx[t])
            pltpu.async_copy(rows[t], acc.at[didx[t]], ssem[t], add=True)
        for b in range(NBA):
            pltpu.make_async_copy(rows[b], acc.at[didx[b]], ssem[b]).wait()
        plsc.subcore_barrier()
        pltpu.sync_copy(acc.at[pl.ds(s * RPS, RPS)],
                        out_hbm.at[c, pl.ds(s * RPS, RPS)])

    return agg_kernel(src, dst, hhat, zeros)


BN = 2000


def _tc_first(degp, x, w):
    def body(degp_ref, x_ref, w_ref, dis_ref, h_ref):
        deg = degp_ref[0, :, 0:1] + degp_ref[1, :, 0:1]
        dis = lax.rsqrt(deg + 1.0)
        dis_ref[...] = dis
        h = jnp.dot(x_ref[...], w_ref[...], preferred_element_type=jnp.float32)
        h_ref[...] = h * dis

    return pl.pallas_call(
        body,
        grid=(N // BN,),
        in_specs=[
            pl.BlockSpec((NC, BN, 128), lambda i: (0, i, 0)),
            pl.BlockSpec((BN, D), lambda i: (i, 0)),
            pl.BlockSpec(w.shape, lambda i: (0, 0)),
        ],
        out_specs=[
            pl.BlockSpec((BN, 1), lambda i: (i, 0)),
            pl.BlockSpec((BN, w.shape[1]), lambda i: (i, 0)),
        ],
        out_shape=[
            jax.ShapeDtypeStruct((N, 1), jnp.float32),
            jax.ShapeDtypeStruct((N, w.shape[1]), jnp.float32),
        ],
    )(degp, x, w)


def _tc_mid(aggp, hhat, dis, b, w):
    f_in = hhat.shape[1]
    f_out = w.shape[1]

    def body(aggp_ref, h_ref, dis_ref, b_ref, w_ref, o_ref):
        t = (aggp_ref[0] + aggp_ref[1] + h_ref[...]) * dis_ref[...] + b_ref[...]
        t = jnp.maximum(t, 0.0)
        o_ref[...] = jnp.dot(t, w_ref[...],
                             preferred_element_type=jnp.float32) * dis_ref[...]

    return pl.pallas_call(
        body,
        grid=(N // BN,),
        in_specs=[
            pl.BlockSpec((NC, BN, f_in), lambda i: (0, i, 0)),
            pl.BlockSpec((BN, f_in), lambda i: (i, 0)),
            pl.BlockSpec((BN, 1), lambda i: (i, 0)),
            pl.BlockSpec((1, f_in), lambda i: (0, 0)),
            pl.BlockSpec((f_in, f_out), lambda i: (0, 0)),
        ],
        out_specs=pl.BlockSpec((BN, f_out), lambda i: (i, 0)),
        out_shape=jax.ShapeDtypeStruct((N, f_out), jnp.float32),
    )(aggp, hhat, dis, b, w)


def _tc_final(aggp, hhat, dis, b):
    f = hhat.shape[1]

    def body(aggp_ref, h_ref, dis_ref, b_ref, o_ref):
        o_ref[...] = ((aggp_ref[0] + aggp_ref[1] + h_ref[...])
                      * dis_ref[...] + b_ref[...])

    return pl.pallas_call(
        body,
        grid=(N // BN,),
        in_specs=[
            pl.BlockSpec((NC, BN, f), lambda i: (0, i, 0)),
            pl.BlockSpec((BN, f), lambda i: (i, 0)),
            pl.BlockSpec((BN, 1), lambda i: (i, 0)),
            pl.BlockSpec((1, f), lambda i: (0, 0)),
        ],
        out_specs=pl.BlockSpec((BN, f), lambda i: (i, 0)),
        out_shape=jax.ShapeDtypeStruct((N, f), jnp.float32),
    )(aggp, hhat, dis, b)


def kernel(x, edge_index, W1, b1, Wm, bm, W2, b2):
    src = edge_index[0]
    dst = edge_index[1]
    H = W1.shape[1]
    C = W2.shape[1]
    CP = 128

    zerosH = jnp.zeros((NP, H), jnp.float32)
    ones128 = jnp.ones((CH, 128), jnp.float32)
    w2p = jnp.zeros((H, CP), jnp.float32).at[:, :C].set(W2)
    b2p = jnp.zeros((CP,), jnp.float32).at[:C].set(b2)

    degp = _sc_degree(dst, zerosH, ones128)
    dis, h1 = _tc_first(degp, x, W1)

    a1 = _sc_aggregate(src, dst, h1, zerosH, H)
    h2 = _tc_mid(a1, h1, dis, b1.reshape(1, H), Wm)

    a2 = _sc_aggregate(src, dst, h2, zerosH, H)
    h3 = _tc_mid(a2, h2, dis, bm.reshape(1, H), w2p)

    a3 = _sc_aggregate(src, dst, h3, zerosH, CP)
    outp = _tc_final(a3, h3, dis, b2p.reshape(1, CP))
    return outp[:, :C]
